# baseline (speedup 1.0000x reference)
import functools

import jax
import jax.numpy as jnp
from jax import lax
from jax.experimental import pallas as pl
from jax.experimental.pallas import tpu as pltpu
from jax.experimental.pallas import tpu_sc as plsc

_GROUP = 128
_K = 5
_MAX_A = 48
_TC_BLK = 1024


def _sc_gather(idx3, table, n_rows, dim):
    info = plsc.get_sparse_core_info()
    nc, ns = info.num_cores, info.num_subcores
    nw = nc * ns
    rpw = n_rows // nw
    ng = rpw // _GROUP
    sup = ng // _K
    cg = _K * _GROUP

    @functools.partial(
        pl.kernel,
        out_type=jax.ShapeDtypeStruct((n_rows, dim), jnp.float32),
        mesh=plsc.VectorSubcoreMesh(core_axis_name="c", subcore_axis_name="s"),
        scratch_types=[
            pltpu.VMEM((ng, _GROUP), jnp.int32),
            pltpu.VMEM((cg, dim), jnp.float32),
            pltpu.VMEM((cg, dim), jnp.float32),
            pltpu.SemaphoreType.DMA,
            pltpu.SemaphoreType.DMA,
        ],
        compiler_params=pltpu.CompilerParams(use_tc_tiling_on_sc=False),
    )
    def gather_k(idx_hbm, table_hbm, out_hbm, idx_v, buf_a, buf_b, sem_a, sem_b):
        c = lax.axis_index("c")
        s = lax.axis_index("s")
        wid = s * nc + c
        base = wid * rpw
        pltpu.sync_copy(idx_hbm.at[wid], idx_v)

        def issue(sc_i, buf, sem):
            for j in range(_K):
                pltpu.async_copy(
                    table_hbm.at[idx_v.at[sc_i * _K + j]],
                    buf.at[pl.ds(j * _GROUP, _GROUP)], sem)

        def drain(sc_i, buf, sem):
            for j in range(_K):
                pltpu.make_async_copy(
                    table_hbm.at[idx_v.at[sc_i * _K + j]],
                    buf.at[pl.ds(j * _GROUP, _GROUP)], sem).wait()

        def write(sc_i, buf):
            pltpu.sync_copy(buf, out_hbm.at[pl.ds(base + sc_i * cg, cg)])

        issue(0, buf_a, sem_a)

        def body(p, carry):
            sa = 2 * p
            sb = 2 * p + 1
            issue(sb, buf_b, sem_b)
            drain(sa, buf_a, sem_a)
            write(sa, buf_a)

            @pl.when(sb + 1 < sup)
            def _():
                issue(sb + 1, buf_a, sem_a)

            drain(sb, buf_b, sem_b)
            write(sb, buf_b)
            return carry

        lax.fori_loop(0, sup // 2, body, 0)

    return gather_k(idx3, table)


_SUB = 64


def _tc_body(k_ref, w_ref, o_ref):
    a_max = jnp.max(k_ref[...]) - 1
    w_max = jnp.max(w_ref[...])
    x_hi = jnp.maximum(w_max, 0.0) + 0.7
    lim = jnp.minimum(jnp.maximum(a_max, 0), _MAX_A)

    def conv_cond(c):
        j, t = c
        return jnp.logical_and(j < lim, t > 1e-8)

    def conv_step(c):
        j, t = c
        return (j + 1, t * (x_hi / (j + 1).astype(jnp.float32)))

    trip, _ = lax.while_loop(conv_cond, conv_step,
                             (jnp.int32(0), jnp.float32(1.0)))

    n_sub = _TC_BLK // _SUB
    for i in range(n_sub):
        rows = pl.ds(i * _SUB, _SUB)
        w = w_ref[rows, :]
        x = jnp.maximum(w, 0.0) + jnp.log1p(jnp.exp(-jnp.abs(w)))

        kin = k_ref[rows, :].astype(jnp.float32)
        a_small = jnp.maximum(kin - 1.0, 0.0)
        grp = lax.broadcasted_iota(jnp.int32, w.shape, 1) // 16
        a = jnp.zeros_like(w)
        for j in range(8):
            a = jnp.where(grp == j, a_small[:, j:j + 1], a)

        def step(j, carry):
            s, t = carry
            jf = j.astype(jnp.float32)
            s = s + jnp.where(a > jf, t, 0.0)
            t = t * (x * (1.0 / (jf + 1.0)))
            return (s, t)

        s, _ = lax.fori_loop(0, trip, step,
                             (jnp.zeros_like(w), jnp.ones_like(w)))
        o_ref[rows, :] = 1.0 - jnp.exp(-x) * s


def _tc_series(kin8, packed, p_rows):
    return pl.pallas_call(
        _tc_body,
        grid=(p_rows // _TC_BLK,),
        in_specs=[
            pl.BlockSpec((_TC_BLK, 8), lambda i: (i, 0)),
            pl.BlockSpec((_TC_BLK, 128), lambda i: (i, 0)),
        ],
        out_specs=pl.BlockSpec((_TC_BLK, 128), lambda i: (i, 0)),
        out_shape=jax.ShapeDtypeStruct((p_rows, 128), jnp.float32),
        compiler_params=pltpu.CompilerParams(
            dimension_semantics=("arbitrary",)),
    )(kin8, packed)


def kernel(problems, behavior_data, W):
    b, l = problems.shape
    dim = W.shape[1]
    n = b * l
    info = plsc.get_sparse_core_info()
    nw = info.num_cores * info.num_subcores

    idx3 = problems.reshape(nw, n // (nw * _GROUP), _GROUP)
    rows = _sc_gather(idx3, W, n, dim)

    p_rows = (n * dim) // 128
    packed = rows.reshape(p_rows, 128)
    kin8 = behavior_data.reshape(p_rows, (128 // dim))
    out = _tc_series(kin8, packed, p_rows)
    return out.reshape(b, l, dim)


# speedup vs baseline: 1.0484x; 1.0484x over previous
import functools

import jax
import jax.numpy as jnp
from jax import lax
from jax.experimental import pallas as pl
from jax.experimental.pallas import tpu as pltpu
from jax.experimental.pallas import tpu_sc as plsc

_GROUP = 128
_K = 5
_MAX_A = 48
_TC_BLK = 1024


def _sc_gather(idx3, table, n_rows, dim):
    info = plsc.get_sparse_core_info()
    nc, ns = info.num_cores, info.num_subcores
    nw = nc * ns
    rpw = n_rows // nw
    ng = rpw // _GROUP
    sup = ng // _K
    cg = _K * _GROUP

    @functools.partial(
        pl.kernel,
        out_type=jax.ShapeDtypeStruct((n_rows, dim), jnp.float32),
        mesh=plsc.VectorSubcoreMesh(core_axis_name="c", subcore_axis_name="s"),
        scratch_types=[
            pltpu.VMEM((ng, _GROUP), jnp.int32),
            pltpu.VMEM((cg, dim), jnp.float32),
            pltpu.VMEM((cg, dim), jnp.float32),
            pltpu.SemaphoreType.DMA,
            pltpu.SemaphoreType.DMA,
        ],
        compiler_params=pltpu.CompilerParams(use_tc_tiling_on_sc=False),
    )
    def gather_k(idx_hbm, table_hbm, out_hbm, idx_v, buf_a, buf_b, sem_a, sem_b):
        c = lax.axis_index("c")
        s = lax.axis_index("s")
        wid = s * nc + c
        base = wid * rpw
        pltpu.sync_copy(idx_hbm.at[wid], idx_v)

        def issue(sc_i, buf, sem):
            for j in range(_K):
                pltpu.async_copy(
                    table_hbm.at[idx_v.at[sc_i * _K + j]],
                    buf.at[pl.ds(j * _GROUP, _GROUP)], sem)

        def drain(sc_i, buf, sem):
            for j in range(_K):
                pltpu.make_async_copy(
                    table_hbm.at[idx_v.at[sc_i * _K + j]],
                    buf.at[pl.ds(j * _GROUP, _GROUP)], sem).wait()

        def write(sc_i, buf):
            pltpu.sync_copy(buf, out_hbm.at[pl.ds(base + sc_i * cg, cg)])

        issue(0, buf_a, sem_a)

        def body(p, carry):
            sa = 2 * p
            sb = 2 * p + 1
            issue(sb, buf_b, sem_b)
            drain(sa, buf_a, sem_a)
            write(sa, buf_a)

            @pl.when(sb + 1 < sup)
            def _():
                issue(sb + 1, buf_a, sem_a)

            drain(sb, buf_b, sem_b)
            write(sb, buf_b)
            return carry

        lax.fori_loop(0, sup // 2, body, 0)

    return gather_k(idx3, table)


_SUB = 64


def _tc_body(k_ref, w_ref, o_ref):
    a_max = jnp.max(k_ref[...])
    w_max = jnp.max(w_ref[...])
    x_hi = jnp.maximum(w_max, 0.0) + 0.7
    lim = jnp.minimum(a_max, jnp.float32(_MAX_A))

    def conv_cond(c):
        j, t = c
        return jnp.logical_and(j < lim, t > 1e-8)

    def conv_step(c):
        j, t = c
        return (j + 1.0, t * (x_hi / (j + 1.0)))

    trip_f, _ = lax.while_loop(conv_cond, conv_step,
                               (jnp.float32(0.0), jnp.float32(1.0)))
    trip = trip_f.astype(jnp.int32)

    n_sub = _TC_BLK // _SUB
    for i in range(n_sub):
        rows = pl.ds(i * _SUB, _SUB)
        w = w_ref[rows, :]
        a = k_ref[rows, :]
        u = jnp.exp(jnp.minimum(w, 80.0))
        x = jnp.where(w > 80.0, w, jnp.log1p(u))
        e = 1.0 / (1.0 + u)

        def step(j, carry):
            s, t = carry
            jf = j.astype(jnp.float32)
            s = s + jnp.where(a > jf, t, 0.0)
            t = t * (x * (1.0 / (jf + 1.0)))
            return (s, t)

        s, _ = lax.fori_loop(0, trip, step,
                             (jnp.zeros_like(w), jnp.ones_like(w)))
        o_ref[rows, :] = 1.0 - e * s


def _tc_series(kin_exp, packed, p_rows):
    return pl.pallas_call(
        _tc_body,
        grid=(p_rows // _TC_BLK,),
        in_specs=[
            pl.BlockSpec((_TC_BLK, 128), lambda i: (i, 0)),
            pl.BlockSpec((_TC_BLK, 128), lambda i: (i, 0)),
        ],
        out_specs=pl.BlockSpec((_TC_BLK, 128), lambda i: (i, 0)),
        out_shape=jax.ShapeDtypeStruct((p_rows, 128), jnp.float32),
        compiler_params=pltpu.CompilerParams(
            dimension_semantics=("arbitrary",)),
    )(kin_exp, packed)


def kernel(problems, behavior_data, W):
    b, l = problems.shape
    dim = W.shape[1]
    n = b * l
    info = plsc.get_sparse_core_info()
    nw = info.num_cores * info.num_subcores

    idx3 = problems.reshape(nw, n // (nw * _GROUP), _GROUP)
    rows = _sc_gather(idx3, W, n, dim)

    p_rows = (n * dim) // 128
    packed = rows.reshape(p_rows, 128)
    a_flat = jnp.maximum(behavior_data.astype(jnp.float32) - 1.0, 0.0)
    kin_exp = jnp.repeat(a_flat.reshape(-1), dim).reshape(p_rows, 128)
    out = _tc_series(kin_exp, packed, p_rows)
    return out.reshape(b, l, dim)
